# baseline (speedup 1.0000x reference)
import functools

import jax
import jax.numpy as jnp
from jax import lax
from jax.experimental import pallas as pl
from jax.experimental.pallas import tpu as pltpu
from jax.experimental.pallas import tpu_sc as plsc

N = 10000
E = 320000
IN = 128
H = 32
NH = 3000

NC = 2
NS = 16
NW = NC * NS
CL = 128
CW = -(-E // (NW * CL))
EP = NW * CW * CL
NP = 10112
RPT = NP // NS
TRASH = N

RB = 1000
NHB = 1000


def _proj_body(x_ref, wl_ref, wr_ref, xl_ref, xr_ref):
    xv = x_ref[...]
    xl_ref[...] = jnp.dot(xv, wl_ref[...], preferred_element_type=jnp.float32)
    xr_ref[...] = jnp.dot(xv, wr_ref[...], preferred_element_type=jnp.float32)


def _mid_body(acc_ref, cnt_ref, xr_ref, wl_ref, wr_ref, b_ref,
              hl_ref, hr_ref):
    a = acc_ref[...]
    c = cnt_ref[...]
    agg = a[0] + a[1]
    cnt = c[0, :, 0:1] + c[1, :, 0:1]
    inv = 1.0 / jnp.maximum(cnt, 1.0)
    h = jnp.maximum(agg * inv + xr_ref[...] + b_ref[...], 0.0)
    hl_ref[...] = jnp.dot(h, wl_ref[...], preferred_element_type=jnp.float32)
    hr_ref[...] = jnp.dot(h, wr_ref[...], preferred_element_type=jnp.float32)


def _fin_body(acc_ref, cnt_ref, hr_ref, b_ref, wf_ref, bf_ref, out_ref):
    a = acc_ref[...]
    c = cnt_ref[...]
    agg = a[0] + a[1]
    cnt = c[0, :, 0:1] + c[1, :, 0:1]
    inv = 1.0 / jnp.maximum(cnt, 1.0)
    h = jnp.maximum(agg * inv + hr_ref[...] + b_ref[...], 0.0)
    out_ref[...] = (
        jnp.dot(h, wf_ref[...], preferred_element_type=jnp.float32)
        + bf_ref[...]
    )


def _edge_loop(table, src_v, bufs, gsems, scatter):
    assert CW % 2 == 1 and CW >= 3
    pltpu.async_copy(table.at[src_v.at[0]], bufs[0], gsems[0])
    pltpu.async_copy(table.at[src_v.at[1]], bufs[1], gsems[1])

    def group(g, carry):
        j0 = 2 * g
        j1 = 2 * g + 1
        pltpu.make_async_copy(table.at[src_v.at[j0]], bufs[0],
                              gsems[0]).wait()
        scatter(bufs[0], j0)
        pltpu.async_copy(table.at[src_v.at[j0 + 2]], bufs[0], gsems[0])
        pltpu.make_async_copy(table.at[src_v.at[j1]], bufs[1],
                              gsems[1]).wait()
        scatter(bufs[1], j1)

        @pl.when(j1 + 2 < CW)
        def _():
            pltpu.async_copy(table.at[src_v.at[j1 + 2]], bufs[1], gsems[1])

        return carry

    lax.fori_loop(0, CW // 2, group, 0)
    jl = CW - 1
    pltpu.make_async_copy(table.at[src_v.at[jl]], bufs[0], gsems[0]).wait()
    scatter(bufs[0], jl)


def _seg_body_counts(table, src2d, dst2d, zacc, zcnt, ones_hbm,
                     out_acc, out_cnt,
                     src_v, dst_v, b0, b1, ones_v,
                     acc_sh, cnt_sh, g0, g1):
    c = lax.axis_index("c")
    s = lax.axis_index("s")
    w = c * NS + s
    pltpu.sync_copy(zacc.at[pl.ds(s * RPT, RPT)],
                    acc_sh.at[pl.ds(s * RPT, RPT)])
    pltpu.sync_copy(zcnt.at[pl.ds(s * RPT, RPT)],
                    cnt_sh.at[pl.ds(s * RPT, RPT)])
    pltpu.sync_copy(ones_hbm, ones_v)
    pltpu.sync_copy(src2d.at[w], src_v)
    pltpu.sync_copy(dst2d.at[w], dst_v)
    plsc.subcore_barrier()
    bufs = [b0, b1]
    gsems = [g0, g1]

    def scatter(rows, j):
        pltpu.sync_copy(rows, acc_sh.at[dst_v.at[j]], add=True)
        pltpu.sync_copy(ones_v, cnt_sh.at[dst_v.at[j]], add=True)

    _edge_loop(table, src_v, bufs, gsems, scatter)
    plsc.subcore_barrier()
    pltpu.sync_copy(acc_sh.at[pl.ds(s * RPT, RPT)],
                    out_acc.at[c].at[pl.ds(s * RPT, RPT)])
    pltpu.sync_copy(cnt_sh.at[pl.ds(s * RPT, RPT)],
                    out_cnt.at[c].at[pl.ds(s * RPT, RPT)])


def _seg_body_plain(table, src2d, dst2d, zacc,
                    out_acc,
                    src_v, dst_v, b0, b1, acc_sh, g0, g1):
    c = lax.axis_index("c")
    s = lax.axis_index("s")
    w = c * NS + s
    pltpu.sync_copy(zacc.at[pl.ds(s * RPT, RPT)],
                    acc_sh.at[pl.ds(s * RPT, RPT)])
    pltpu.sync_copy(src2d.at[w], src_v)
    pltpu.sync_copy(dst2d.at[w], dst_v)
    plsc.subcore_barrier()
    bufs = [b0, b1]
    gsems = [g0, g1]

    def scatter(rows, j):
        pltpu.sync_copy(rows, acc_sh.at[dst_v.at[j]], add=True)

    _edge_loop(table, src_v, bufs, gsems, scatter)
    plsc.subcore_barrier()
    pltpu.sync_copy(acc_sh.at[pl.ds(s * RPT, RPT)],
                    out_acc.at[c].at[pl.ds(s * RPT, RPT)])


def _sc_mesh():
    return plsc.VectorSubcoreMesh(core_axis_name="c", subcore_axis_name="s",
                                  num_cores=NC, num_subcores=NS)


@functools.lru_cache(maxsize=None)
def _segsum_counts_kernel():
    return pl.kernel(
        _seg_body_counts,
        out_type=[jax.ShapeDtypeStruct((NC, NP, H), jnp.float32),
                  jax.ShapeDtypeStruct((NC, NP, 16), jnp.float32)],
        mesh=_sc_mesh(),
        scratch_types=(
            [pltpu.VMEM((CW, CL), jnp.int32)] * 2
            + [pltpu.VMEM((CL, H), jnp.float32)] * 2
            + [pltpu.VMEM((CL, 16), jnp.float32)]
            + [pltpu.VMEM_SHARED((NP, H), jnp.float32),
               pltpu.VMEM_SHARED((NP, 16), jnp.float32)]
            + [pltpu.SemaphoreType.DMA] * 2
        ),
        compiler_params=pltpu.CompilerParams(use_tc_tiling_on_sc=False),
    )


@functools.lru_cache(maxsize=None)
def _segsum_plain_kernel():
    return pl.kernel(
        _seg_body_plain,
        out_type=[jax.ShapeDtypeStruct((NC, NP, H), jnp.float32)],
        mesh=_sc_mesh(),
        scratch_types=(
            [pltpu.VMEM((CW, CL), jnp.int32)] * 2
            + [pltpu.VMEM((CL, H), jnp.float32)] * 2
            + [pltpu.VMEM_SHARED((NP, H), jnp.float32)]
            + [pltpu.SemaphoreType.DMA] * 2
        ),
        compiler_params=pltpu.CompilerParams(use_tc_tiling_on_sc=False),
    )


@jax.jit
def kernel(x, edge_index, W1l, W1r, b1, W2l, W2r, b2, Wf, bf):
    src = edge_index[0].astype(jnp.int32)
    dst = edge_index[1].astype(jnp.int32)
    ppw = (EP - E) // NW
    src2d = jnp.concatenate(
        [src.reshape(NW, E // NW),
         jnp.zeros((NW, ppw), jnp.int32)], axis=1).reshape(NW, CW, CL)
    pad_dst = jnp.broadcast_to(
        TRASH + jnp.arange(ppw, dtype=jnp.int32) % (NP - N), (NW, ppw))
    dst2d = jnp.concatenate(
        [dst.reshape(NW, E // NW), pad_dst], axis=1).reshape(NW, CW, CL)
    zacc = jnp.zeros((NP, H), jnp.float32)
    zcnt = jnp.zeros((NP, 16), jnp.float32)
    ones = jnp.ones((CL, 16), jnp.float32)
    b1r = b1.reshape(1, H)
    b2r = b2.reshape(1, H)
    bfr = bf.reshape(1, NH)

    xl, xr = pl.pallas_call(
        _proj_body,
        grid=(N // RB,),
        in_specs=[
            pl.BlockSpec((RB, IN), lambda i: (i, 0)),
            pl.BlockSpec((IN, H), lambda i: (0, 0)),
            pl.BlockSpec((IN, H), lambda i: (0, 0)),
        ],
        out_specs=[pl.BlockSpec((RB, H), lambda i: (i, 0))] * 2,
        out_shape=[jax.ShapeDtypeStruct((N, H), jnp.float32)] * 2,
    )(x, W1l, W1r)

    acc1, cnt = _segsum_counts_kernel()(xl, src2d, dst2d, zacc, zcnt, ones)

    h1l, h1r = pl.pallas_call(
        _mid_body,
        grid=(N // RB,),
        in_specs=[
            pl.BlockSpec((NC, RB, H), lambda i: (0, i, 0)),
            pl.BlockSpec((NC, RB, 16), lambda i: (0, i, 0)),
            pl.BlockSpec((RB, H), lambda i: (i, 0)),
            pl.BlockSpec((H, H), lambda i: (0, 0)),
            pl.BlockSpec((H, H), lambda i: (0, 0)),
            pl.BlockSpec((1, H), lambda i: (0, 0)),
        ],
        out_specs=[pl.BlockSpec((RB, H), lambda i: (i, 0))] * 2,
        out_shape=[jax.ShapeDtypeStruct((N, H), jnp.float32)] * 2,
    )(acc1, cnt, xr, W2l, W2r, b1r)

    (acc2,) = _segsum_plain_kernel()(h1l, src2d, dst2d, zacc)

    out = pl.pallas_call(
        _fin_body,
        grid=(N // RB,),
        in_specs=[
            pl.BlockSpec((NC, RB, H), lambda i: (0, i, 0)),
            pl.BlockSpec((NC, RB, 16), lambda i: (0, i, 0)),
            pl.BlockSpec((RB, H), lambda i: (i, 0)),
            pl.BlockSpec((1, H), lambda i: (0, 0)),
            pl.BlockSpec((H, NH), lambda i: (0, 0)),
            pl.BlockSpec((1, NH), lambda i: (0, 0)),
        ],
        out_specs=pl.BlockSpec((RB, NH), lambda i: (i, 0)),
        out_shape=jax.ShapeDtypeStruct((N, NH), jnp.float32),
    )(acc2, cnt, h1r, b2r, Wf, bfr)

    return out


# speedup vs baseline: 1.0590x; 1.0590x over previous
import functools

import jax
import jax.numpy as jnp
from jax import lax
from jax.experimental import pallas as pl
from jax.experimental.pallas import tpu as pltpu
from jax.experimental.pallas import tpu_sc as plsc

N = 10000
E = 320000
IN = 128
H = 32
NH = 3000

NC = 2
NS = 16
NW = NC * NS
CL = 128
CW = -(-E // (NW * CL))
EP = NW * CW * CL
NP = 10112
RPT = NP // NS
TRASH = N

RB = 1000
NHB = 1000


def _proj_body(x_ref, wl_ref, wr_ref, xl_ref, xr_ref):
    xv = x_ref[...]
    xl_ref[...] = jnp.dot(xv, wl_ref[...], preferred_element_type=jnp.float32)
    xr_ref[...] = jnp.dot(xv, wr_ref[...], preferred_element_type=jnp.float32)


def _mid_body(acc_ref, cnt_ref, xr_ref, wl_ref, wr_ref, b_ref,
              hl_ref, hr_ref):
    a = acc_ref[...]
    c = cnt_ref[...]
    agg = a[0] + a[1]
    cnt = c[0, :, 0:1] + c[1, :, 0:1]
    inv = 1.0 / jnp.maximum(cnt, 1.0)
    h = jnp.maximum(agg * inv + xr_ref[...] + b_ref[...], 0.0)
    hl_ref[...] = jnp.dot(h, wl_ref[...], preferred_element_type=jnp.float32)
    hr_ref[...] = jnp.dot(h, wr_ref[...], preferred_element_type=jnp.float32)


def _fin_body(acc_ref, cnt_ref, hr_ref, b_ref, wf_ref, bf_ref, out_ref):
    a = acc_ref[...]
    c = cnt_ref[...]
    agg = a[0] + a[1]
    cnt = c[0, :, 0:1] + c[1, :, 0:1]
    inv = 1.0 / jnp.maximum(cnt, 1.0)
    h = jnp.maximum(agg * inv + hr_ref[...] + b_ref[...], 0.0)
    out_ref[...] = (
        jnp.dot(h, wf_ref[...], preferred_element_type=jnp.float32)
        + bf_ref[...]
    )


def _edge_loop(table, src_v, bufs, gsems, scatter):
    assert CW % 3 == 1 and CW >= 7
    for b in range(3):
        pltpu.async_copy(table.at[src_v.at[b]], bufs[b], gsems[b])

    def group(g, carry):
        for k in range(3):
            j = 3 * g + k
            pltpu.make_async_copy(table.at[src_v.at[j]], bufs[k],
                                  gsems[k]).wait()
            scatter(bufs[k], j)
            pltpu.async_copy(table.at[src_v.at[j + 3]], bufs[k], gsems[k])
        return carry

    lax.fori_loop(0, (CW - 4) // 3, group, 0)
    j0 = CW - 4
    pltpu.make_async_copy(table.at[src_v.at[j0]], bufs[j0 % 3],
                          gsems[j0 % 3]).wait()
    scatter(bufs[j0 % 3], j0)
    pltpu.async_copy(table.at[src_v.at[CW - 1]], bufs[j0 % 3],
                     gsems[j0 % 3])
    for j in range(CW - 3, CW):
        pltpu.make_async_copy(table.at[src_v.at[j]], bufs[j % 3],
                              gsems[j % 3]).wait()
        scatter(bufs[j % 3], j)


def _seg_body_counts(table, src2d, dst2d, zacc, zcnt, ones_hbm,
                     out_acc, out_cnt,
                     src_v, dst_v, b0, b1, b2, ones_v,
                     acc_sh, cnt_sh, g0, g1, g2):
    c = lax.axis_index("c")
    s = lax.axis_index("s")
    w = c * NS + s
    pltpu.sync_copy(zacc.at[pl.ds(s * RPT, RPT)],
                    acc_sh.at[pl.ds(s * RPT, RPT)])
    pltpu.sync_copy(zcnt.at[pl.ds(s * RPT, RPT)],
                    cnt_sh.at[pl.ds(s * RPT, RPT)])
    pltpu.sync_copy(ones_hbm, ones_v)
    pltpu.sync_copy(src2d.at[w], src_v)
    pltpu.sync_copy(dst2d.at[w], dst_v)
    plsc.subcore_barrier()
    bufs = [b0, b1, b2]
    gsems = [g0, g1, g2]

    def scatter(rows, j):
        pltpu.sync_copy(rows, acc_sh.at[dst_v.at[j]], add=True)
        pltpu.sync_copy(ones_v, cnt_sh.at[dst_v.at[j]], add=True)

    _edge_loop(table, src_v, bufs, gsems, scatter)
    plsc.subcore_barrier()
    pltpu.sync_copy(acc_sh.at[pl.ds(s * RPT, RPT)],
                    out_acc.at[c].at[pl.ds(s * RPT, RPT)])
    pltpu.sync_copy(cnt_sh.at[pl.ds(s * RPT, RPT)],
                    out_cnt.at[c].at[pl.ds(s * RPT, RPT)])


def _seg_body_plain(table, src2d, dst2d, zacc,
                    out_acc,
                    src_v, dst_v, b0, b1, b2, acc_sh, g0, g1, g2):
    c = lax.axis_index("c")
    s = lax.axis_index("s")
    w = c * NS + s
    pltpu.sync_copy(zacc.at[pl.ds(s * RPT, RPT)],
                    acc_sh.at[pl.ds(s * RPT, RPT)])
    pltpu.sync_copy(src2d.at[w], src_v)
    pltpu.sync_copy(dst2d.at[w], dst_v)
    plsc.subcore_barrier()
    bufs = [b0, b1, b2]
    gsems = [g0, g1, g2]

    def scatter(rows, j):
        pltpu.sync_copy(rows, acc_sh.at[dst_v.at[j]], add=True)

    _edge_loop(table, src_v, bufs, gsems, scatter)
    plsc.subcore_barrier()
    pltpu.sync_copy(acc_sh.at[pl.ds(s * RPT, RPT)],
                    out_acc.at[c].at[pl.ds(s * RPT, RPT)])


def _sc_mesh():
    return plsc.VectorSubcoreMesh(core_axis_name="c", subcore_axis_name="s",
                                  num_cores=NC, num_subcores=NS)


@functools.lru_cache(maxsize=None)
def _segsum_counts_kernel():
    return pl.kernel(
        _seg_body_counts,
        out_type=[jax.ShapeDtypeStruct((NC, NP, H), jnp.float32),
                  jax.ShapeDtypeStruct((NC, NP, 16), jnp.float32)],
        mesh=_sc_mesh(),
        scratch_types=(
            [pltpu.VMEM((CW, CL), jnp.int32)] * 2
            + [pltpu.VMEM((CL, H), jnp.float32)] * 3
            + [pltpu.VMEM((CL, 16), jnp.float32)]
            + [pltpu.VMEM_SHARED((NP, H), jnp.float32),
               pltpu.VMEM_SHARED((NP, 16), jnp.float32)]
            + [pltpu.SemaphoreType.DMA] * 3
        ),
        compiler_params=pltpu.CompilerParams(use_tc_tiling_on_sc=False),
    )


@functools.lru_cache(maxsize=None)
def _segsum_plain_kernel():
    return pl.kernel(
        _seg_body_plain,
        out_type=[jax.ShapeDtypeStruct((NC, NP, H), jnp.float32)],
        mesh=_sc_mesh(),
        scratch_types=(
            [pltpu.VMEM((CW, CL), jnp.int32)] * 2
            + [pltpu.VMEM((CL, H), jnp.float32)] * 3
            + [pltpu.VMEM_SHARED((NP, H), jnp.float32)]
            + [pltpu.SemaphoreType.DMA] * 3
        ),
        compiler_params=pltpu.CompilerParams(use_tc_tiling_on_sc=False),
    )


@jax.jit
def kernel(x, edge_index, W1l, W1r, b1, W2l, W2r, b2, Wf, bf):
    src = edge_index[0].astype(jnp.int32)
    dst = edge_index[1].astype(jnp.int32)
    ppw = (EP - E) // NW
    src2d = jnp.concatenate(
        [src.reshape(NW, E // NW),
         jnp.zeros((NW, ppw), jnp.int32)], axis=1).reshape(NW, CW, CL)
    pad_dst = jnp.broadcast_to(
        TRASH + jnp.arange(ppw, dtype=jnp.int32) % (NP - N), (NW, ppw))
    dst2d = jnp.concatenate(
        [dst.reshape(NW, E // NW), pad_dst], axis=1).reshape(NW, CW, CL)
    zacc = jnp.zeros((NP, H), jnp.float32)
    zcnt = jnp.zeros((NP, 16), jnp.float32)
    ones = jnp.ones((CL, 16), jnp.float32)
    b1r = b1.reshape(1, H)
    b2r = b2.reshape(1, H)
    bfr = bf.reshape(1, NH)

    xl, xr = pl.pallas_call(
        _proj_body,
        grid=(N // RB,),
        in_specs=[
            pl.BlockSpec((RB, IN), lambda i: (i, 0)),
            pl.BlockSpec((IN, H), lambda i: (0, 0)),
            pl.BlockSpec((IN, H), lambda i: (0, 0)),
        ],
        out_specs=[pl.BlockSpec((RB, H), lambda i: (i, 0))] * 2,
        out_shape=[jax.ShapeDtypeStruct((N, H), jnp.float32)] * 2,
    )(x, W1l, W1r)

    acc1, cnt = _segsum_counts_kernel()(xl, src2d, dst2d, zacc, zcnt, ones)

    h1l, h1r = pl.pallas_call(
        _mid_body,
        grid=(N // RB,),
        in_specs=[
            pl.BlockSpec((NC, RB, H), lambda i: (0, i, 0)),
            pl.BlockSpec((NC, RB, 16), lambda i: (0, i, 0)),
            pl.BlockSpec((RB, H), lambda i: (i, 0)),
            pl.BlockSpec((H, H), lambda i: (0, 0)),
            pl.BlockSpec((H, H), lambda i: (0, 0)),
            pl.BlockSpec((1, H), lambda i: (0, 0)),
        ],
        out_specs=[pl.BlockSpec((RB, H), lambda i: (i, 0))] * 2,
        out_shape=[jax.ShapeDtypeStruct((N, H), jnp.float32)] * 2,
    )(acc1, cnt, xr, W2l, W2r, b1r)

    (acc2,) = _segsum_plain_kernel()(h1l, src2d, dst2d, zacc)

    out = pl.pallas_call(
        _fin_body,
        grid=(N // RB,),
        in_specs=[
            pl.BlockSpec((NC, RB, H), lambda i: (0, i, 0)),
            pl.BlockSpec((NC, RB, 16), lambda i: (0, i, 0)),
            pl.BlockSpec((RB, H), lambda i: (i, 0)),
            pl.BlockSpec((1, H), lambda i: (0, 0)),
            pl.BlockSpec((H, NH), lambda i: (0, 0)),
            pl.BlockSpec((1, NH), lambda i: (0, 0)),
        ],
        out_specs=pl.BlockSpec((RB, NH), lambda i: (i, 0)),
        out_shape=jax.ShapeDtypeStruct((N, NH), jnp.float32),
    )(acc2, cnt, h1r, b2r, Wf, bfr)

    return out


# speedup vs baseline: 1.0803x; 1.0201x over previous
import functools

import jax
import jax.numpy as jnp
from jax import lax
from jax.experimental import pallas as pl
from jax.experimental.pallas import tpu as pltpu
from jax.experimental.pallas import tpu_sc as plsc

N = 10000
E = 320000
IN = 128
H = 32
NH = 3000

NC = 2
NS = 16
NW = NC * NS
CL = 128
CW = -(-E // (NW * CL))
EP = NW * CW * CL
NP = 10112
RPT = NP // NS
TRASH = N

RB = 1000
NHB = 1000


def _proj_body(x_ref, wl_ref, wr_ref, xl_ref, xr_ref):
    xv = x_ref[...]
    xl_ref[...] = jnp.dot(xv, wl_ref[...], preferred_element_type=jnp.float32)
    xr_ref[...] = jnp.dot(xv, wr_ref[...], preferred_element_type=jnp.float32)


def _mid_body(acc_ref, cnt_ref, xr_ref, wl_ref, wr_ref, b_ref,
              hl_ref, hr_ref):
    a = acc_ref[...]
    c = cnt_ref[...]
    agg = a[0] + a[1]
    cnt = c[0, :, 0:1] + c[1, :, 0:1]
    inv = 1.0 / jnp.maximum(cnt, 1.0)
    h = jnp.maximum(agg * inv + xr_ref[...] + b_ref[...], 0.0)
    hl_ref[...] = jnp.dot(h, wl_ref[...], preferred_element_type=jnp.float32)
    hr_ref[...] = jnp.dot(h, wr_ref[...], preferred_element_type=jnp.float32)


def _fin_body(acc_ref, cnt_ref, hr_ref, b_ref, wf_ref, bf_ref, out_ref):
    a = acc_ref[...]
    c = cnt_ref[...]
    agg = a[0] + a[1]
    cnt = c[0, :, 0:1] + c[1, :, 0:1]
    inv = 1.0 / jnp.maximum(cnt, 1.0)
    h = jnp.maximum(agg * inv + hr_ref[...] + b_ref[...], 0.0)
    out_ref[...] = (
        jnp.dot(h, wf_ref[...], preferred_element_type=jnp.float32)
        + bf_ref[...]
    )


def _edge_loop(table, src_v, bufs, gsems, scatter):
    D = len(bufs)
    assert CW >= 2 * D
    for b in range(D):
        pltpu.async_copy(table.at[src_v.at[b]], bufs[b], gsems[b])
    ng = (CW - D) // D

    def group(g, carry):
        for k in range(D):
            j = D * g + k
            pltpu.make_async_copy(table.at[src_v.at[j]], bufs[k],
                                  gsems[k]).wait()
            scatter(bufs[k], j)
            pltpu.async_copy(table.at[src_v.at[j + D]], bufs[k], gsems[k])
        return carry

    lax.fori_loop(0, ng, group, 0)
    for j in range(D * ng, CW):
        pltpu.make_async_copy(table.at[src_v.at[j]], bufs[j % D],
                              gsems[j % D]).wait()
        scatter(bufs[j % D], j)
        if j + D < CW:
            pltpu.async_copy(table.at[src_v.at[j + D]], bufs[j % D],
                             gsems[j % D])


def _seg_body_counts(table, src2d, dst2d, zacc, zcnt, ones_hbm,
                     out_acc, out_cnt,
                     src_v, dst_v, b0, b1, b2, b3, ones_v,
                     acc_sh, cnt_sh, g0, g1, g2, g3):
    c = lax.axis_index("c")
    s = lax.axis_index("s")
    w = c * NS + s
    pltpu.sync_copy(zacc.at[pl.ds(s * RPT, RPT)],
                    acc_sh.at[pl.ds(s * RPT, RPT)])
    pltpu.sync_copy(zcnt.at[pl.ds(s * RPT, RPT)],
                    cnt_sh.at[pl.ds(s * RPT, RPT)])
    pltpu.sync_copy(ones_hbm, ones_v)
    pltpu.sync_copy(src2d.at[w], src_v)
    pltpu.sync_copy(dst2d.at[w], dst_v)
    plsc.subcore_barrier()
    bufs = [b0, b1, b2, b3]
    gsems = [g0, g1, g2, g3]

    def scatter(rows, j):
        pltpu.sync_copy(rows, acc_sh.at[dst_v.at[j]], add=True)
        pltpu.sync_copy(ones_v, cnt_sh.at[dst_v.at[j]], add=True)

    _edge_loop(table, src_v, bufs, gsems, scatter)
    plsc.subcore_barrier()
    pltpu.sync_copy(acc_sh.at[pl.ds(s * RPT, RPT)],
                    out_acc.at[c].at[pl.ds(s * RPT, RPT)])
    pltpu.sync_copy(cnt_sh.at[pl.ds(s * RPT, RPT)],
                    out_cnt.at[c].at[pl.ds(s * RPT, RPT)])


def _seg_body_plain(table, src2d, dst2d, zacc,
                    out_acc,
                    src_v, dst_v, b0, b1, b2, b3, acc_sh, g0, g1, g2, g3):
    c = lax.axis_index("c")
    s = lax.axis_index("s")
    w = c * NS + s
    pltpu.sync_copy(zacc.at[pl.ds(s * RPT, RPT)],
                    acc_sh.at[pl.ds(s * RPT, RPT)])
    pltpu.sync_copy(src2d.at[w], src_v)
    pltpu.sync_copy(dst2d.at[w], dst_v)
    plsc.subcore_barrier()
    bufs = [b0, b1, b2, b3]
    gsems = [g0, g1, g2, g3]

    def scatter(rows, j):
        pltpu.sync_copy(rows, acc_sh.at[dst_v.at[j]], add=True)

    _edge_loop(table, src_v, bufs, gsems, scatter)
    plsc.subcore_barrier()
    pltpu.sync_copy(acc_sh.at[pl.ds(s * RPT, RPT)],
                    out_acc.at[c].at[pl.ds(s * RPT, RPT)])


def _sc_mesh():
    return plsc.VectorSubcoreMesh(core_axis_name="c", subcore_axis_name="s",
                                  num_cores=NC, num_subcores=NS)


@functools.lru_cache(maxsize=None)
def _segsum_counts_kernel():
    return pl.kernel(
        _seg_body_counts,
        out_type=[jax.ShapeDtypeStruct((NC, NP, H), jnp.float32),
                  jax.ShapeDtypeStruct((NC, NP, 16), jnp.float32)],
        mesh=_sc_mesh(),
        scratch_types=(
            [pltpu.VMEM((CW, CL), jnp.int32)] * 2
            + [pltpu.VMEM((CL, H), jnp.float32)] * 4
            + [pltpu.VMEM((CL, 16), jnp.float32)]
            + [pltpu.VMEM_SHARED((NP, H), jnp.float32),
               pltpu.VMEM_SHARED((NP, 16), jnp.float32)]
            + [pltpu.SemaphoreType.DMA] * 4
        ),
        compiler_params=pltpu.CompilerParams(use_tc_tiling_on_sc=False),
    )


@functools.lru_cache(maxsize=None)
def _segsum_plain_kernel():
    return pl.kernel(
        _seg_body_plain,
        out_type=[jax.ShapeDtypeStruct((NC, NP, H), jnp.float32)],
        mesh=_sc_mesh(),
        scratch_types=(
            [pltpu.VMEM((CW, CL), jnp.int32)] * 2
            + [pltpu.VMEM((CL, H), jnp.float32)] * 4
            + [pltpu.VMEM_SHARED((NP, H), jnp.float32)]
            + [pltpu.SemaphoreType.DMA] * 4
        ),
        compiler_params=pltpu.CompilerParams(use_tc_tiling_on_sc=False),
    )


@jax.jit
def kernel(x, edge_index, W1l, W1r, b1, W2l, W2r, b2, Wf, bf):
    src = edge_index[0].astype(jnp.int32)
    dst = edge_index[1].astype(jnp.int32)
    ppw = (EP - E) // NW
    src2d = jnp.concatenate(
        [src.reshape(NW, E // NW),
         jnp.zeros((NW, ppw), jnp.int32)], axis=1).reshape(NW, CW, CL)
    pad_dst = jnp.broadcast_to(
        TRASH + jnp.arange(ppw, dtype=jnp.int32) % (NP - N), (NW, ppw))
    dst2d = jnp.concatenate(
        [dst.reshape(NW, E // NW), pad_dst], axis=1).reshape(NW, CW, CL)
    zacc = jnp.zeros((NP, H), jnp.float32)
    zcnt = jnp.zeros((NP, 16), jnp.float32)
    ones = jnp.ones((CL, 16), jnp.float32)
    b1r = b1.reshape(1, H)
    b2r = b2.reshape(1, H)
    bfr = bf.reshape(1, NH)

    xl, xr = pl.pallas_call(
        _proj_body,
        grid=(N // RB,),
        in_specs=[
            pl.BlockSpec((RB, IN), lambda i: (i, 0)),
            pl.BlockSpec((IN, H), lambda i: (0, 0)),
            pl.BlockSpec((IN, H), lambda i: (0, 0)),
        ],
        out_specs=[pl.BlockSpec((RB, H), lambda i: (i, 0))] * 2,
        out_shape=[jax.ShapeDtypeStruct((N, H), jnp.float32)] * 2,
    )(x, W1l, W1r)

    acc1, cnt = _segsum_counts_kernel()(xl, src2d, dst2d, zacc, zcnt, ones)

    h1l, h1r = pl.pallas_call(
        _mid_body,
        grid=(N // RB,),
        in_specs=[
            pl.BlockSpec((NC, RB, H), lambda i: (0, i, 0)),
            pl.BlockSpec((NC, RB, 16), lambda i: (0, i, 0)),
            pl.BlockSpec((RB, H), lambda i: (i, 0)),
            pl.BlockSpec((H, H), lambda i: (0, 0)),
            pl.BlockSpec((H, H), lambda i: (0, 0)),
            pl.BlockSpec((1, H), lambda i: (0, 0)),
        ],
        out_specs=[pl.BlockSpec((RB, H), lambda i: (i, 0))] * 2,
        out_shape=[jax.ShapeDtypeStruct((N, H), jnp.float32)] * 2,
    )(acc1, cnt, xr, W2l, W2r, b1r)

    (acc2,) = _segsum_plain_kernel()(h1l, src2d, dst2d, zacc)

    out = pl.pallas_call(
        _fin_body,
        grid=(N // RB,),
        in_specs=[
            pl.BlockSpec((NC, RB, H), lambda i: (0, i, 0)),
            pl.BlockSpec((NC, RB, 16), lambda i: (0, i, 0)),
            pl.BlockSpec((RB, H), lambda i: (i, 0)),
            pl.BlockSpec((1, H), lambda i: (0, 0)),
            pl.BlockSpec((H, NH), lambda i: (0, 0)),
            pl.BlockSpec((1, NH), lambda i: (0, 0)),
        ],
        out_specs=pl.BlockSpec((RB, NH), lambda i: (i, 0)),
        out_shape=jax.ShapeDtypeStruct((N, NH), jnp.float32),
    )(acc2, cnt, h1r, b2r, Wf, bfr)

    return out


# speedup vs baseline: 1.0896x; 1.0087x over previous
import functools

import jax
import jax.numpy as jnp
from jax import lax
from jax.experimental import pallas as pl
from jax.experimental.pallas import tpu as pltpu
from jax.experimental.pallas import tpu_sc as plsc

N = 10000
E = 320000
IN = 128
H = 32
NH = 3000

NC = 2
NS = 16
NW = NC * NS
CL = 128
CW = -(-E // (NW * CL))
EP = NW * CW * CL
NP = 10112
RPT = NP // NS
TRASH = N

RB = 1000
NHB = 1000


def _proj_body(x_ref, wl_ref, wr_ref, xl_ref, xr_ref):
    xv = x_ref[...]
    xl_ref[...] = jnp.dot(xv, wl_ref[...], preferred_element_type=jnp.float32)
    xr_ref[...] = jnp.dot(xv, wr_ref[...], preferred_element_type=jnp.float32)


def _mid_body(acc_ref, cnt_ref, xr_ref, wl_ref, wr_ref, b_ref,
              hl_ref, hr_ref):
    a = acc_ref[...]
    c = cnt_ref[...]
    agg = a[0] + a[1]
    cnt = c[0, :, 0:1] + c[1, :, 0:1]
    inv = 1.0 / jnp.maximum(cnt, 1.0)
    h = jnp.maximum(agg * inv + xr_ref[...] + b_ref[...], 0.0)
    hl_ref[...] = jnp.dot(h, wl_ref[...], preferred_element_type=jnp.float32)
    hr_ref[...] = jnp.dot(h, wr_ref[...], preferred_element_type=jnp.float32)


def _fin_body(acc_ref, cnt_ref, hr_ref, b_ref, wf_ref, bf_ref, out_ref):
    a = acc_ref[...]
    c = cnt_ref[...]
    agg = a[0] + a[1]
    cnt = c[0, :, 0:1] + c[1, :, 0:1]
    inv = 1.0 / jnp.maximum(cnt, 1.0)
    h = jnp.maximum(agg * inv + hr_ref[...] + b_ref[...], 0.0)
    out_ref[...] = (
        jnp.dot(h, wf_ref[...], preferred_element_type=jnp.float32)
        + bf_ref[...]
    )


def _edge_loop(table, src_v, bufs, gsems, scatter):
    D = len(bufs)
    assert CW >= 2 * D
    for b in range(D):
        pltpu.async_copy(table.at[src_v.at[b]], bufs[b], gsems[b])
    ng = (CW - D) // D

    def group(g, carry):
        for k in range(D):
            j = D * g + k
            pltpu.make_async_copy(table.at[src_v.at[j]], bufs[k],
                                  gsems[k]).wait()
            scatter(bufs[k], j)
            pltpu.async_copy(table.at[src_v.at[j + D]], bufs[k], gsems[k])
        return carry

    lax.fori_loop(0, ng, group, 0)
    for j in range(D * ng, CW):
        pltpu.make_async_copy(table.at[src_v.at[j]], bufs[j % D],
                              gsems[j % D]).wait()
        scatter(bufs[j % D], j)
        if j + D < CW:
            pltpu.async_copy(table.at[src_v.at[j + D]], bufs[j % D],
                             gsems[j % D])


def _seg_body_counts(table, src2d, dst2d, zacc, zcnt, ones_hbm,
                     out_acc, out_cnt,
                     src_v, dst_v, b0, b1, b2, b3, b4, b5, ones_v,
                     acc_sh, cnt_sh, g0, g1, g2, g3, g4, g5):
    c = lax.axis_index("c")
    s = lax.axis_index("s")
    w = c * NS + s
    pltpu.sync_copy(zacc.at[pl.ds(s * RPT, RPT)],
                    acc_sh.at[pl.ds(s * RPT, RPT)])
    pltpu.sync_copy(zcnt.at[pl.ds(s * RPT, RPT)],
                    cnt_sh.at[pl.ds(s * RPT, RPT)])
    pltpu.sync_copy(ones_hbm, ones_v)
    pltpu.sync_copy(src2d.at[w], src_v)
    pltpu.sync_copy(dst2d.at[w], dst_v)
    plsc.subcore_barrier()
    bufs = [b0, b1, b2, b3, b4, b5]
    gsems = [g0, g1, g2, g3, g4, g5]

    def scatter(rows, j):
        pltpu.sync_copy(rows, acc_sh.at[dst_v.at[j]], add=True)
        pltpu.sync_copy(ones_v, cnt_sh.at[dst_v.at[j]], add=True)

    _edge_loop(table, src_v, bufs, gsems, scatter)
    plsc.subcore_barrier()
    pltpu.sync_copy(acc_sh.at[pl.ds(s * RPT, RPT)],
                    out_acc.at[c].at[pl.ds(s * RPT, RPT)])
    pltpu.sync_copy(cnt_sh.at[pl.ds(s * RPT, RPT)],
                    out_cnt.at[c].at[pl.ds(s * RPT, RPT)])


def _seg_body_plain(table, src2d, dst2d, zacc,
                    out_acc,
                    src_v, dst_v, b0, b1, b2, b3, b4, b5, acc_sh,
                    g0, g1, g2, g3, g4, g5):
    c = lax.axis_index("c")
    s = lax.axis_index("s")
    w = c * NS + s
    pltpu.sync_copy(zacc.at[pl.ds(s * RPT, RPT)],
                    acc_sh.at[pl.ds(s * RPT, RPT)])
    pltpu.sync_copy(src2d.at[w], src_v)
    pltpu.sync_copy(dst2d.at[w], dst_v)
    plsc.subcore_barrier()
    bufs = [b0, b1, b2, b3, b4, b5]
    gsems = [g0, g1, g2, g3, g4, g5]

    def scatter(rows, j):
        pltpu.sync_copy(rows, acc_sh.at[dst_v.at[j]], add=True)

    _edge_loop(table, src_v, bufs, gsems, scatter)
    plsc.subcore_barrier()
    pltpu.sync_copy(acc_sh.at[pl.ds(s * RPT, RPT)],
                    out_acc.at[c].at[pl.ds(s * RPT, RPT)])


def _sc_mesh():
    return plsc.VectorSubcoreMesh(core_axis_name="c", subcore_axis_name="s",
                                  num_cores=NC, num_subcores=NS)


@functools.lru_cache(maxsize=None)
def _segsum_counts_kernel():
    return pl.kernel(
        _seg_body_counts,
        out_type=[jax.ShapeDtypeStruct((NC, NP, H), jnp.float32),
                  jax.ShapeDtypeStruct((NC, NP, 16), jnp.float32)],
        mesh=_sc_mesh(),
        scratch_types=(
            [pltpu.VMEM((CW, CL), jnp.int32)] * 2
            + [pltpu.VMEM((CL, H), jnp.float32)] * 6
            + [pltpu.VMEM((CL, 16), jnp.float32)]
            + [pltpu.VMEM_SHARED((NP, H), jnp.float32),
               pltpu.VMEM_SHARED((NP, 16), jnp.float32)]
            + [pltpu.SemaphoreType.DMA] * 6
        ),
        compiler_params=pltpu.CompilerParams(use_tc_tiling_on_sc=False),
    )


@functools.lru_cache(maxsize=None)
def _segsum_plain_kernel():
    return pl.kernel(
        _seg_body_plain,
        out_type=[jax.ShapeDtypeStruct((NC, NP, H), jnp.float32)],
        mesh=_sc_mesh(),
        scratch_types=(
            [pltpu.VMEM((CW, CL), jnp.int32)] * 2
            + [pltpu.VMEM((CL, H), jnp.float32)] * 6
            + [pltpu.VMEM_SHARED((NP, H), jnp.float32)]
            + [pltpu.SemaphoreType.DMA] * 6
        ),
        compiler_params=pltpu.CompilerParams(use_tc_tiling_on_sc=False),
    )


@jax.jit
def kernel(x, edge_index, W1l, W1r, b1, W2l, W2r, b2, Wf, bf):
    src = edge_index[0].astype(jnp.int32)
    dst = edge_index[1].astype(jnp.int32)
    ppw = (EP - E) // NW
    src2d = jnp.concatenate(
        [src.reshape(NW, E // NW),
         jnp.zeros((NW, ppw), jnp.int32)], axis=1).reshape(NW, CW, CL)
    pad_dst = jnp.broadcast_to(
        TRASH + jnp.arange(ppw, dtype=jnp.int32) % (NP - N), (NW, ppw))
    dst2d = jnp.concatenate(
        [dst.reshape(NW, E // NW), pad_dst], axis=1).reshape(NW, CW, CL)
    zacc = jnp.zeros((NP, H), jnp.float32)
    zcnt = jnp.zeros((NP, 16), jnp.float32)
    ones = jnp.ones((CL, 16), jnp.float32)
    b1r = b1.reshape(1, H)
    b2r = b2.reshape(1, H)
    bfr = bf.reshape(1, NH)

    xl, xr = pl.pallas_call(
        _proj_body,
        grid=(N // RB,),
        in_specs=[
            pl.BlockSpec((RB, IN), lambda i: (i, 0)),
            pl.BlockSpec((IN, H), lambda i: (0, 0)),
            pl.BlockSpec((IN, H), lambda i: (0, 0)),
        ],
        out_specs=[pl.BlockSpec((RB, H), lambda i: (i, 0))] * 2,
        out_shape=[jax.ShapeDtypeStruct((N, H), jnp.float32)] * 2,
    )(x, W1l, W1r)

    acc1, cnt = _segsum_counts_kernel()(xl, src2d, dst2d, zacc, zcnt, ones)

    h1l, h1r = pl.pallas_call(
        _mid_body,
        grid=(N // RB,),
        in_specs=[
            pl.BlockSpec((NC, RB, H), lambda i: (0, i, 0)),
            pl.BlockSpec((NC, RB, 16), lambda i: (0, i, 0)),
            pl.BlockSpec((RB, H), lambda i: (i, 0)),
            pl.BlockSpec((H, H), lambda i: (0, 0)),
            pl.BlockSpec((H, H), lambda i: (0, 0)),
            pl.BlockSpec((1, H), lambda i: (0, 0)),
        ],
        out_specs=[pl.BlockSpec((RB, H), lambda i: (i, 0))] * 2,
        out_shape=[jax.ShapeDtypeStruct((N, H), jnp.float32)] * 2,
    )(acc1, cnt, xr, W2l, W2r, b1r)

    (acc2,) = _segsum_plain_kernel()(h1l, src2d, dst2d, zacc)

    out = pl.pallas_call(
        _fin_body,
        grid=(N // RB,),
        in_specs=[
            pl.BlockSpec((NC, RB, H), lambda i: (0, i, 0)),
            pl.BlockSpec((NC, RB, 16), lambda i: (0, i, 0)),
            pl.BlockSpec((RB, H), lambda i: (i, 0)),
            pl.BlockSpec((1, H), lambda i: (0, 0)),
            pl.BlockSpec((H, NH), lambda i: (0, 0)),
            pl.BlockSpec((1, NH), lambda i: (0, 0)),
        ],
        out_specs=pl.BlockSpec((RB, NH), lambda i: (i, 0)),
        out_shape=jax.ShapeDtypeStruct((N, NH), jnp.float32),
    )(acc2, cnt, h1r, b2r, Wf, bfr)

    return out
